# baseline (speedup 1.0000x reference)
import functools

import jax
import jax.numpy as jnp
from jax import lax
from jax.experimental import pallas as pl
from jax.experimental.pallas import tpu as pltpu
from jax.experimental.pallas import tpu_sc as plsc

VOCAB = 100000
D = 64
S = 50
Q = 4096
NC = 2
NS = 16
NW = NC * NS
CH = 128

_mesh = plsc.VectorSubcoreMesh(core_axis_name="c", subcore_axis_name="s")


@functools.partial(
    pl.kernel,
    mesh=_mesh,
    out_type=jax.ShapeDtypeStruct((S, D // 8, NW, 8, CH), jnp.float32),
    scratch_types=[
        pltpu.VMEM((S, CH), jnp.int32),
        [pltpu.VMEM((CH, CH), jnp.float32)] * 2,
        [pltpu.VMEM((D // 8, 8, CH + 1), jnp.float32)] * 2,
        [pltpu.SemaphoreType.DMA] * 2,
        [pltpu.SemaphoreType.DMA] * 2,
    ],
    compiler_params=pltpu.CompilerParams(use_tc_tiling_on_sc=False,
                                         needs_layout_passes=False),
)
def _gather(table_hbm, idxt_hbm, out_hbm, idx_v, chunks, tbufs, gsems, ssems):
    wid = lax.axis_index("s") * NC + lax.axis_index("c")
    pltpu.sync_copy(idxt_hbm.at[:, pl.ds(wid * CH, CH)], idx_v)

    zv = jnp.minimum(idx_v[0, pl.ds(0, 16)], 0)
    iot = lax.iota(jnp.int32, 16) + zv
    ftv = [(iot + 16 * k) >> 3 for k in range(4)]
    fiv = [(iot + 16 * k) & 7 for k in range(4)]

    def ig(s, b):
        pltpu.async_copy(table_hbm.at[idx_v.at[s]], chunks[b],
                         gsems[b])

    def wg(s, b):
        pltpu.make_async_copy(table_hbm.at[idx_v.at[s]],
                              chunks[b], gsems[b]).wait()

    def iscat(s, b):
        pltpu.async_copy(tbufs[b].at[:, :, pl.ds(0, CH)],
                         out_hbm.at[s, :, wid], ssems[b])

    def wscat(s, b):
        pltpu.make_async_copy(tbufs[b].at[:, :, pl.ds(0, CH)],
                              out_hbm.at[s, :, wid], ssems[b]).wait()

    def transpose(b):
        def qstep(qo, carry):
            prev = []
            for dq in range(8):
                q = qo * 8 + dq
                qv = zv + q
                cur = [(k, qv, chunks[b][q, pl.ds(k * 16, 16)])
                       for k in range(4)]
                for k, qvp, vals in prev:
                    plsc.store_scatter(tbufs[b], [ftv[k], fiv[k], qvp], vals)
                prev = cur
            for k, qvp, vals in prev:
                plsc.store_scatter(tbufs[b], [ftv[k], fiv[k], qvp], vals)
            return carry
        lax.fori_loop(0, CH // 8, qstep, 0)

    ig(0, 0)
    ig(1, 1)
    for b in range(2):
        wg(b, b)
        transpose(b)
        iscat(b, b)
        ig(b + 2, b)

    def body(p, carry):
        for b in range(2):
            s = 2 * p + b
            wg(s, b)
            wscat(s - 2, b)
            transpose(b)
            iscat(s, b)
            ig(s + 2, b)
        return carry

    lax.fori_loop(1, S // 2 - 1, body, 0)

    for b in range(2):
        s = S - 2 + b
        wg(s, b)
        wscat(s - 2, b)
        transpose(b)
        iscat(s, b)
    for b in range(2):
        wscat(S - 2 + b, b)


def kernel(input_ids, embedding_matrix):
    idx_t = input_ids.T
    tblp = jnp.pad(embedding_matrix, ((0, 0), (0, CH - D)))
    out5 = _gather(tblp, idx_t)
    return out5.transpose(2, 4, 0, 1, 3).reshape(Q, S, D)


# speedup vs baseline: 1.0336x; 1.0336x over previous
import functools

import jax
import jax.numpy as jnp
from jax import lax
from jax.experimental import pallas as pl
from jax.experimental.pallas import tpu as pltpu
from jax.experimental.pallas import tpu_sc as plsc

VOCAB = 100000
D = 64
S = 50
Q = 4096
NC = 2
NS = 16
NW = NC * NS
CH = 128

_mesh = plsc.VectorSubcoreMesh(core_axis_name="c", subcore_axis_name="s")


@functools.partial(
    pl.kernel,
    mesh=_mesh,
    out_type=jax.ShapeDtypeStruct((S, D // 8, NW, 8, CH), jnp.float32),
    scratch_types=[
        pltpu.VMEM((S, CH), jnp.int32),
        [pltpu.VMEM((CH, D), jnp.float32)] * 4,
        [pltpu.VMEM((D // 8, 8, CH + 1), jnp.float32)] * 4,
        [pltpu.SemaphoreType.DMA] * 4,
        [pltpu.SemaphoreType.DMA] * 4,
    ],
    compiler_params=pltpu.CompilerParams(use_tc_tiling_on_sc=False,
                                         needs_layout_passes=False),
)
def _gather(table_hbm, idxt_hbm, out_hbm, idx_v, chunks, tbufs, gsems, ssems):
    wid = lax.axis_index("s") * NC + lax.axis_index("c")
    pltpu.sync_copy(idxt_hbm.at[:, pl.ds(wid * CH, CH)], idx_v)

    zv = jnp.minimum(idx_v[0, pl.ds(0, 16)], 0)
    iot = lax.iota(jnp.int32, 16) + zv
    ftv = [(iot + 16 * k) >> 3 for k in range(4)]
    fiv = [(iot + 16 * k) & 7 for k in range(4)]

    def ig(s, b):
        pltpu.async_copy(table_hbm.at[idx_v.at[s]], chunks[b], gsems[b])

    def wg(s, b):
        pltpu.make_async_copy(table_hbm.at[idx_v.at[s]], chunks[b],
                              gsems[b]).wait()

    def iscat(s, b):
        pltpu.async_copy(tbufs[b].at[:, :, pl.ds(0, CH)],
                         out_hbm.at[s, :, wid], ssems[b])

    def wscat(s, b):
        pltpu.make_async_copy(tbufs[b].at[:, :, pl.ds(0, CH)],
                              out_hbm.at[s, :, wid], ssems[b]).wait()

    def transpose(b):
        def qstep(qo, carry):
            prev = []
            for dq in range(8):
                q = qo * 8 + dq
                qv = zv + q
                cur = [(k, qv, chunks[b][q, pl.ds(k * 16, 16)])
                       for k in range(4)]
                for k, qvp, vals in prev:
                    plsc.store_scatter(tbufs[b], [ftv[k], fiv[k], qvp], vals)
                prev = cur
            for k, qvp, vals in prev:
                plsc.store_scatter(tbufs[b], [ftv[k], fiv[k], qvp], vals)
            return carry
        lax.fori_loop(0, CH // 8, qstep, 0)

    for b in range(4):
        ig(b, b)
    for b in range(4):
        wg(b, b)
        transpose(b)
        iscat(b, b)
        ig(b + 4, b)

    def body(j, carry):
        for b in range(4):
            s = 4 * j + 4 + b
            wg(s, b)
            wscat(s - 4, b)
            transpose(b)
            iscat(s, b)
            ig(s + 4, b)
        return carry

    lax.fori_loop(0, (S - 8) // 4, body, 0)

    for b in range(4):
        s = S - 6 + b
        wg(s, b)
        wscat(s - 4, b)
        transpose(b)
        iscat(s, b)
        if s + 4 < S:
            ig(s + 4, b)
    for b in range(2):
        s = S - 2 + b
        wg(s, b)
        wscat(s - 4, b)
        transpose(b)
        iscat(s, b)
    for b in range(2, 4):
        wscat(S - 6 + b, b)
    for b in range(2):
        wscat(S - 2 + b, b)


def kernel(input_ids, embedding_matrix):
    idx_t = input_ids.T
    out5 = _gather(embedding_matrix, idx_t)
    return out5.transpose(2, 4, 0, 1, 3).reshape(Q, S, D)


# speedup vs baseline: 1.0337x; 1.0002x over previous
import functools

import jax
import jax.numpy as jnp
from jax import lax
from jax.experimental import pallas as pl
from jax.experimental.pallas import tpu as pltpu
from jax.experimental.pallas import tpu_sc as plsc

VOCAB = 100000
D = 64
S = 50
Q = 4096
NC = 2
NS = 16
NW = NC * NS
CH = 128

_mesh = plsc.VectorSubcoreMesh(core_axis_name="c", subcore_axis_name="s")


@functools.partial(
    pl.kernel,
    mesh=_mesh,
    out_type=jax.ShapeDtypeStruct((S, D // 8, NW, 8, CH), jnp.float32),
    scratch_types=[
        pltpu.VMEM((S, CH), jnp.int32),
        [pltpu.VMEM((CH, D), jnp.float32)] * 2,
        [pltpu.VMEM((D // 8, 8, CH + 1), jnp.float32)] * 2,
        [pltpu.SemaphoreType.DMA] * 2,
        [pltpu.SemaphoreType.DMA] * 2,
    ],
    compiler_params=pltpu.CompilerParams(use_tc_tiling_on_sc=False,
                                         needs_layout_passes=False),
)
def _gather(table_hbm, idxt_hbm, out_hbm, idx_v, chunks, tbufs, gsems, ssems):
    wid = lax.axis_index("s") * NC + lax.axis_index("c")
    pltpu.sync_copy(idxt_hbm.at[:, pl.ds(wid * CH, CH)], idx_v)

    zv = jnp.minimum(idx_v[0, pl.ds(0, 16)], 0)
    iot = lax.iota(jnp.int32, 16) + zv
    ftv = [(iot + 16 * k) >> 3 for k in range(4)]
    fiv = [(iot + 16 * k) & 7 for k in range(4)]

    def ig(s, b):
        pltpu.async_copy(table_hbm.at[idx_v.at[s]], chunks[b], gsems[b])

    def wg(s, b):
        pltpu.make_async_copy(table_hbm.at[idx_v.at[s]], chunks[b],
                              gsems[b]).wait()

    def iscat(s, b):
        pltpu.async_copy(tbufs[b].at[:, :, pl.ds(0, CH)],
                         out_hbm.at[s, :, wid], ssems[b])

    def wscat(s, b):
        pltpu.make_async_copy(tbufs[b].at[:, :, pl.ds(0, CH)],
                              out_hbm.at[s, :, wid], ssems[b]).wait()

    def transpose(b):
        def qstep(qo, carry):
            prev = []
            for dq in range(8):
                q = qo * 8 + dq
                qv = zv + q
                cur = [(k, qv, chunks[b][q, pl.ds(k * 16, 16)])
                       for k in range(4)]
                for k, qvp, vals in prev:
                    plsc.store_scatter(tbufs[b], [ftv[k], fiv[k], qvp], vals)
                prev = cur
            for k, qvp, vals in prev:
                plsc.store_scatter(tbufs[b], [ftv[k], fiv[k], qvp], vals)
            return carry
        lax.fori_loop(0, CH // 8, qstep, 0)

    ig(0, 0)
    ig(1, 1)
    for b in range(2):
        wg(b, b)
        transpose(b)
        iscat(b, b)
        ig(b + 2, b)

    def body(p, carry):
        for b in range(2):
            s = 2 * p + b
            wg(s, b)
            wscat(s - 2, b)
            transpose(b)
            iscat(s, b)
            ig(s + 2, b)
        return carry

    lax.fori_loop(1, S // 2 - 1, body, 0)

    for b in range(2):
        s = S - 2 + b
        wg(s, b)
        wscat(s - 2, b)
        transpose(b)
        iscat(s, b)
    for b in range(2):
        wscat(S - 2 + b, b)


def kernel(input_ids, embedding_matrix):
    idx_t = input_ids.T
    out5 = _gather(embedding_matrix, idx_t)
    return out5.transpose(2, 4, 0, 1, 3).reshape(Q, S, D)


# speedup vs baseline: 1.0975x; 1.0617x over previous
import functools

import jax
import jax.numpy as jnp
from jax import lax
from jax.experimental import pallas as pl
from jax.experimental.pallas import tpu as pltpu
from jax.experimental.pallas import tpu_sc as plsc

VOCAB = 100000
D = 64
S = 50
Q = 4096
NC = 2
NS = 16
NW = NC * NS
CH = 128

_mesh = plsc.VectorSubcoreMesh(core_axis_name="c", subcore_axis_name="s")


@functools.partial(
    pl.kernel,
    mesh=_mesh,
    out_type=jax.ShapeDtypeStruct((S, D // 8, NW, 8, CH), jnp.float32),
    scratch_types=[
        pltpu.VMEM((S, CH), jnp.int32),
        [pltpu.VMEM((CH, D), jnp.float32)] * 2,
        [pltpu.VMEM((D // 8, 8, CH + 1), jnp.float32)] * 2,
        [pltpu.SemaphoreType.DMA] * 2,
        [pltpu.SemaphoreType.DMA] * 2,
    ],
    compiler_params=pltpu.CompilerParams(use_tc_tiling_on_sc=False,
                                         needs_layout_passes=False),
)
def _gather(table_hbm, idxt_hbm, out_hbm, idx_v, chunks, tbufs, gsems, ssems):
    wid = lax.axis_index("s") * NC + lax.axis_index("c")
    pltpu.sync_copy(idxt_hbm.at[:, pl.ds(wid * CH, CH)], idx_v)

    zv = jnp.minimum(idx_v[0, pl.ds(0, 16)], 0)
    iot = lax.iota(jnp.int32, 16) + zv
    ftv = [(iot + 16 * k) >> 3 for k in range(4)]
    fiv = [(iot + 16 * k) & 7 for k in range(4)]

    def ig(s, b):
        pltpu.async_copy(table_hbm.at[idx_v.at[s]], chunks[b], gsems[b])

    def wg(s, b):
        pltpu.make_async_copy(table_hbm.at[idx_v.at[s]], chunks[b],
                              gsems[b]).wait()

    def iscat(s, b):
        pltpu.async_copy(tbufs[b].at[:, :, pl.ds(0, CH)],
                         out_hbm.at[s, :, wid], ssems[b])

    def wscat(s, b):
        pltpu.make_async_copy(tbufs[b].at[:, :, pl.ds(0, CH)],
                              out_hbm.at[s, :, wid], ssems[b]).wait()

    def transpose(b):
        def qstep(qo, carry):
            prev = []
            for dq in range(8):
                q = qo * 8 + dq
                qv = zv + q
                cur = [(k, qv, chunks[b][q, pl.ds(k * 16, 16)])
                       for k in range(4)]
                for k, qvp, vals in prev:
                    plsc.store_scatter(tbufs[b], [ftv[k], fiv[k], qvp], vals)
                prev = cur
            for k, qvp, vals in prev:
                plsc.store_scatter(tbufs[b], [ftv[k], fiv[k], qvp], vals)
            return carry
        lax.fori_loop(0, CH // 8, qstep, 0)

    ig(0, 0)
    ig(1, 1)
    for b in range(2):
        wg(b, b)
        transpose(b)
        iscat(b, b)
        ig(b + 2, b)

    def body(p, carry):
        for b in range(2):
            s = 2 * p + b
            wg(s, b)
            wscat(s - 2, b)
            transpose(b)
            iscat(s, b)
            ig(s + 2, b)
        return carry

    lax.fori_loop(1, S // 2 - 1, body, 0)

    for b in range(2):
        s = S - 2 + b
        wg(s, b)
        wscat(s - 2, b)
        transpose(b)
        iscat(s, b)
    for b in range(2):
        wscat(S - 2 + b, b)


def kernel(input_ids, embedding_matrix):
    idx_t = input_ids.T * 2
    tblp = jnp.pad(embedding_matrix, ((0, 0), (0, CH - D)))
    out5 = _gather(tblp.reshape(2 * VOCAB, D), idx_t)
    return out5.transpose(2, 4, 0, 1, 3).reshape(Q, S, D)
